# baseline (speedup 1.0000x reference)
import functools

import jax
import jax.numpy as jnp
from jax import lax
from jax.experimental import pallas as pl
from jax.experimental.pallas import tpu as pltpu
from jax.experimental.pallas import tpu_sc as plsc


def _sc_gather(idx, x, chunk, hbm_chunks):
    (B,) = idx.shape
    V, D = x.shape
    info = plsc.get_sparse_core_info()
    nc, ns = info.num_cores, info.num_subcores
    nw = nc * ns
    b_per_w = B // nw
    n_full = b_per_w // chunk
    tail = b_per_w - n_full * chunk
    if n_full % 2:
        n_full -= 1
        tail += chunk
    assert B % nw == 0 and chunk % 8 == 0 and tail % 8 == 0 and tail <= chunk
    assert hbm_chunks % 2 == 0 and hbm_chunks + 2 <= n_full

    mesh = plsc.VectorSubcoreMesh(core_axis_name="c", subcore_axis_name="s")

    @functools.partial(
        pl.kernel,
        mesh=mesh,
        out_type=jax.ShapeDtypeStruct((B, D), x.dtype),
        scratch_types=[
            pltpu.VMEM_SHARED((V, D), x.dtype),
            pltpu.VMEM((b_per_w,), jnp.int32),
            pltpu.VMEM((chunk, D), x.dtype),
            pltpu.VMEM((chunk, D), x.dtype),
            pltpu.SemaphoreType.DMA,
            pltpu.SemaphoreType.DMA,
            pltpu.SemaphoreType.DMA,
            pltpu.SemaphoreType.DMA,
            pltpu.SemaphoreType.DMA,
        ],
    )
    def k(idx_hbm, x_hbm, out_hbm, x_sp, idx_v, rows0, rows1, g0, g1, s0, s1, st):
        sid = lax.axis_index("s")
        wid = sid * nc + lax.axis_index("c")
        base = wid * b_per_w
        rows = (rows0, rows1)
        gsem = (g0, g1)
        ssem = (s0, s1)

        v_per_s = (V // ns) // 8 * 8
        stage = pltpu.make_async_copy(
            x_hbm.at[pl.ds(sid * v_per_s, v_per_s)],
            x_sp.at[pl.ds(sid * v_per_s, v_per_s)],
            st,
        )
        stage.start()
        v_rem = V - ns * v_per_s
        if v_rem:
            stage_rem = pltpu.make_async_copy(
                x_hbm.at[pl.ds(ns * v_per_s, v_rem)],
                x_sp.at[pl.ds(ns * v_per_s, v_rem)],
                st,
            )

            @pl.when(sid == 0)
            def _():
                stage_rem.start()

        pltpu.sync_copy(idx_hbm.at[pl.ds(base, b_per_w)], idx_v)

        def gather_hbm(j, b):
            return pltpu.make_async_copy(
                x_hbm.at[idx_v.at[pl.ds(j * chunk, chunk)]], rows[b], gsem[b]
            )

        def gather(j, b, sz=chunk):
            src = x_sp if b == 0 else x_hbm
            return pltpu.make_async_copy(
                src.at[idx_v.at[pl.ds(j * chunk, sz)]],
                rows[b].at[pl.ds(0, sz)],
                gsem[b],
            )

        def scatter(j, b, sz=chunk):
            return pltpu.make_async_copy(
                rows[b].at[pl.ds(0, sz)],
                out_hbm.at[pl.ds(base + j * chunk, sz)],
                ssem[b],
            )

        gather_hbm(0, 0).start()
        gather_hbm(1, 1).start()

        def body1(i, carry):
            for b in range(2):
                j = 2 * i + b
                gather_hbm(j, b).wait()
                scatter(j, b).start()

                @pl.when(j + 2 < hbm_chunks)
                def _():
                    scatter(j, b).wait()
                    gather_hbm(j + 2, b).start()

            return carry

        lax.fori_loop(0, hbm_chunks // 2, body1, 0)

        stage.wait()
        if v_rem:

            @pl.when(sid == 0)
            def _():
                stage_rem.wait()

        plsc.subcore_barrier()

        scatter(hbm_chunks - 2, 0).wait()
        gather(hbm_chunks, 0).start()
        scatter(hbm_chunks - 1, 1).wait()
        gather(hbm_chunks + 1, 1).start()

        def body2(i2, carry):
            for b in range(2):
                j = hbm_chunks + 2 * i2 + b
                gather(j, b).wait()
                scatter(j, b).start()

                @pl.when(j + 2 < n_full)
                def _():
                    scatter(j, b).wait()
                    gather(j + 2, b).start()

                if tail:
                    @pl.when(j + 2 == n_full)
                    def _():
                        scatter(j, b).wait()
                        gather(n_full, 0, tail).start()

            return carry

        lax.fori_loop(0, (n_full - hbm_chunks) // 2, body2, 0)
        if tail:
            gather(n_full, 0, tail).wait()
            scatter(n_full, 0, tail).start()
            scatter(n_full - 1, 1).wait()
            scatter(n_full, 0, tail).wait()
        else:
            scatter(n_full - 2, 0).wait()
            scatter(n_full - 1, 1).wait()

    return k(idx, x)


def kernel(edge_index, x):
    idx = edge_index[1]
    return _sc_gather(idx, x, chunk=80, hbm_chunks=8)


# speedup vs baseline: 1.4824x; 1.4824x over previous
import functools

import jax
import jax.numpy as jnp
from jax import lax
from jax.experimental import pallas as pl
from jax.experimental.pallas import tpu as pltpu
from jax.experimental.pallas import tpu_sc as plsc


def _sc_gather(idx, x, chunk, hbm_chunks):
    (B,) = idx.shape
    V, D = x.shape
    info = plsc.get_sparse_core_info()
    nc, ns = info.num_cores, info.num_subcores
    nw = nc * ns
    b_per_w = B // nw
    n_full = b_per_w // chunk
    tail = b_per_w - n_full * chunk
    if n_full % 2:
        n_full -= 1
        tail += chunk
    assert B % nw == 0 and chunk % 8 == 0 and tail % 8 == 0 and tail <= chunk
    assert hbm_chunks % 2 == 0 and hbm_chunks + 2 <= n_full

    mesh = plsc.VectorSubcoreMesh(core_axis_name="c", subcore_axis_name="s")

    @functools.partial(
        pl.kernel,
        mesh=mesh,
        out_type=jax.ShapeDtypeStruct((B, D), x.dtype),
        scratch_types=[
            pltpu.VMEM_SHARED((V, D), x.dtype),
            pltpu.VMEM((b_per_w,), jnp.int32),
            pltpu.VMEM((chunk, D), x.dtype),
            pltpu.VMEM((chunk, D), x.dtype),
            pltpu.SemaphoreType.DMA,
            pltpu.SemaphoreType.DMA,
            pltpu.SemaphoreType.DMA,
            pltpu.SemaphoreType.DMA,
            pltpu.SemaphoreType.DMA,
        ],
    )
    def k(idx_hbm, x_hbm, out_hbm, x_sp, idx_v, rows0, rows1, g0, g1, s0, s1, st):
        sid = lax.axis_index("s")
        wid = sid * nc + lax.axis_index("c")
        base = wid * b_per_w
        rows = (rows0, rows1)
        gsem = (g0, g1)
        ssem = (s0, s1)

        v_per_s = (V // ns) // 8 * 8
        stage = pltpu.make_async_copy(
            x_hbm.at[pl.ds(sid * v_per_s, v_per_s)],
            x_sp.at[pl.ds(sid * v_per_s, v_per_s)],
            st,
        )
        stage.start()
        v_rem = V - ns * v_per_s
        if v_rem:
            stage_rem = pltpu.make_async_copy(
                x_hbm.at[pl.ds(ns * v_per_s, v_rem)],
                x_sp.at[pl.ds(ns * v_per_s, v_rem)],
                st,
            )

            @pl.when(sid == 0)
            def _():
                stage_rem.start()

        pltpu.sync_copy(idx_hbm.at[pl.ds(base, b_per_w)], idx_v)

        def gather_hbm(j, b):
            return pltpu.make_async_copy(
                x_hbm.at[idx_v.at[pl.ds(j * chunk, chunk)]], rows[b], gsem[b]
            )

        def gather(j, b, sz=chunk):
            return pltpu.make_async_copy(
                x_sp.at[idx_v.at[pl.ds(j * chunk, sz)]],
                rows[b].at[pl.ds(0, sz)],
                gsem[b],
            )

        def scatter(j, b, sz=chunk):
            return pltpu.make_async_copy(
                rows[b].at[pl.ds(0, sz)],
                out_hbm.at[pl.ds(base + j * chunk, sz)],
                ssem[b],
            )

        gather_hbm(0, 0).start()
        gather_hbm(1, 1).start()

        def body1(i, carry):
            for b in range(2):
                j = 2 * i + b
                gather_hbm(j, b).wait()
                scatter(j, b).start()

                @pl.when(j + 2 < hbm_chunks)
                def _():
                    scatter(j, b).wait()
                    gather_hbm(j + 2, b).start()

            return carry

        lax.fori_loop(0, hbm_chunks // 2, body1, 0)

        stage.wait()
        if v_rem:

            @pl.when(sid == 0)
            def _():
                stage_rem.wait()

        plsc.subcore_barrier()

        scatter(hbm_chunks - 2, 0).wait()
        gather(hbm_chunks, 0).start()
        scatter(hbm_chunks - 1, 1).wait()
        gather(hbm_chunks + 1, 1).start()

        def body2(i2, carry):
            for b in range(2):
                j = hbm_chunks + 2 * i2 + b
                gather(j, b).wait()
                scatter(j, b).start()

                @pl.when(j + 2 < n_full)
                def _():
                    scatter(j, b).wait()
                    gather(j + 2, b).start()

                if tail:
                    @pl.when(j + 2 == n_full)
                    def _():
                        scatter(j, b).wait()
                        gather(n_full, 0, tail).start()

            return carry

        lax.fori_loop(0, (n_full - hbm_chunks) // 2, body2, 0)
        if tail:
            gather(n_full, 0, tail).wait()
            scatter(n_full, 0, tail).start()
            scatter(n_full - 1, 1).wait()
            scatter(n_full, 0, tail).wait()
        else:
            scatter(n_full - 2, 0).wait()
            scatter(n_full - 1, 1).wait()

    return k(idx, x)


def kernel(edge_index, x):
    idx = edge_index[1]
    return _sc_gather(idx, x, chunk=80, hbm_chunks=8)
